# baseline (speedup 1.0000x reference)
import functools

import jax
import jax.numpy as jnp
from jax import lax
from jax.experimental import pallas as pl
from jax.experimental.pallas import tpu as pltpu
from jax.experimental.pallas import tpu_sc as plsc

_B, _NC, _L = 16, 16, 512
_MO, _A, _NID = 1024, 128, 64
_OUT = 352
_TOK = _B * _NC * _L
_NCELL = _B * _NC
_NCORES, _NSUB = 2, 16
_NW = _NCORES * _NSUB
_CH = 128
_NROW = _TOK // _CH
_RPW = _NROW // _NW


def _tab_body(a_ref, w_ref, nid_ref, tw1_ref, tw2_ref, nl_ref, nr_ref):
    a = a_ref[0]
    w = w_ref[...]
    tw1_ref[0] = jnp.dot(a, w[:_A, :], preferred_element_type=jnp.float32,
                         precision=lax.Precision.HIGHEST)
    tw2_ref[0] = jnp.dot(a, w[_A:, :], preferred_element_type=jnp.float32,
                         precision=lax.Precision.HIGHEST)
    nid = nid_ref[0]
    z32 = jnp.zeros((_NC * _NID, 32), jnp.float32)
    z64 = jnp.zeros((_NC * _NID, _NID), jnp.float32)
    nl_ref[0] = jnp.concatenate([nid[:, 32:64], z64, nid[:, 0:32]], axis=1)
    nr_ref[0] = jnp.concatenate([z32, nid, z32], axis=1)


def _sc_body(tw1, tw2, nl, nr, i1h, i2h, j1h, j2h, outa, outn,
             i1v, i2v, j1v, j2v, g1, g2, h1, h2, s1, s2, s3, s4):
    wid = lax.axis_index("s") * _NCORES + lax.axis_index("c")
    row0 = wid * _RPW
    b = wid // 2

    pltpu.sync_copy(i1h.at[pl.ds(row0, _RPW)], i1v)
    pltpu.sync_copy(i2h.at[pl.ds(row0, _RPW)], i2v)
    pltpu.sync_copy(j1h.at[pl.ds(row0, _RPW)], j1v)
    pltpu.sync_copy(j2h.at[pl.ds(row0, _RPW)], j2v)

    def bias_row(r, carry):
        cell = (row0 + r) // 4
        aoff = b * _MO
        noff = cell * _NID
        for j in range(8):
            sl = pl.ds(j * 16, 16)
            i1v[r, sl] = i1v[r, sl] + aoff
            i2v[r, sl] = i2v[r, sl] + aoff
            j1v[r, sl] = j1v[r, sl] + noff
            j2v[r, sl] = j2v[r, sl] + noff
        return carry

    lax.fori_loop(0, _RPW, bias_row, 0)

    def chunk(c, carry):
        base = (row0 + c) * _CH
        cp1 = pltpu.async_copy(tw1.at[i1v.at[c]], g1, s1)
        cp2 = pltpu.async_copy(tw2.at[i2v.at[c]], g2, s2)
        cp3 = pltpu.async_copy(nl.at[j1v.at[c]], h1, s3)
        cp4 = pltpu.async_copy(nr.at[j2v.at[c]], h2, s4)
        cp1.wait()
        cp2.wait()

        def add_a(r, cc):
            for j in range(8):
                sl = pl.ds(j * 16, 16)
                g1[r, sl] = g1[r, sl] + g2[r, sl]
            return cc

        lax.fori_loop(0, _CH, add_a, 0)
        pltpu.sync_copy(g1, outa.at[pl.ds(base, _CH)])
        cp3.wait()
        cp4.wait()

        def add_n(r, cc):
            for j in range(8):
                sl = pl.ds(j * 16, 16)
                h1[r, sl] = h1[r, sl] + h2[r, sl]
            return cc

        lax.fori_loop(0, _CH, add_n, 0)
        pltpu.sync_copy(h1, outn.at[pl.ds(base, _CH)])
        return carry

    lax.fori_loop(0, _RPW, chunk, 0)


def _prep_body(wc_ref, ba_ref, bc_ref, bf_ref, ph_ref, te_ref,
               r_ref, fp_ref, s_ref):
    f32 = jnp.float32
    wrow = jnp.concatenate([
        jnp.zeros((8, 128), f32), wc_ref[...],
        jnp.zeros((8, 192), f32)], axis=1)
    brow = jnp.concatenate([
        ba_ref[...], bc_ref[...], jnp.zeros((192,), f32)])[None, :]
    r_ref[...] = jnp.concatenate([te_ref[...], wrow, brow], axis=0)
    fcat = jnp.concatenate([bf_ref[...], bf_ref[...]])[None, :]
    pcat = jnp.concatenate([ph_ref[...], ph_ref[...]])[None, :]
    fp_ref[...] = jnp.concatenate([fcat, pcat], axis=0)
    lane = lax.broadcasted_iota(jnp.int32, (2, 64), 1)
    row = lax.broadcasted_iota(jnp.int32, (2, 64), 0)
    s_ref[...] = jnp.where((lane < 32) == (row == 0), 1.0, 0.0)


def _time_body(tpt_ref, tpf_ref, fp_ref, s_ref, ht_ref):
    maxt = jnp.max(tpf_ref[0])
    tpts = jnp.dot(tpt_ref[0], s_ref[...], preferred_element_type=jnp.float32,
                   precision=lax.Precision.HIGHEST)
    ht_ref[0] = jnp.cos((maxt - tpts) * fp_ref[0:1, :] + fp_ref[1:2, :])


def _fin_body(oa_ref, on_ref, ht_ref, tty_ref, cf_ref, r_ref, out_ref):
    tty = tty_ref[0]
    oh = (tty == jnp.arange(3, dtype=jnp.int32)[None, :]).astype(jnp.float32)
    cf = cf_ref[0]
    ones = jnp.ones((_L, 1), jnp.float32)
    z = jnp.concatenate([oh, cf, ones], axis=1)
    y = jnp.dot(z, r_ref[...],
                preferred_element_type=jnp.float32)
    zl = jnp.zeros((_L, 32), jnp.float32)
    h12 = jnp.concatenate([zl, ht_ref[0], zl], axis=1)
    onr = on_ref[0]
    lane = lax.broadcasted_iota(jnp.int32, (1, 128), 1)
    add1 = jnp.where(lane >= 96, onr, h12)
    out_ref[0, :, 0:128] = y[:, 0:128] + oa_ref[0]
    out_ref[0, :, 128:256] = y[:, 128:256] + add1
    out_ref[0, :, 256:352] = y[:, 256:352] + onr[:, 0:96]


def kernel(token_pair_idx, token_pair_time, token_types, attr_feats_lookup,
           coord_feats, idx_in_lookup, n_id_lookup,
           W_attr, b_attr, W_coor, b_coor, basis_freq, phase, type_emb):
    f32 = jnp.float32
    nid3 = n_id_lookup.reshape(_B, _NC * _NID, _NID)
    tw1, tw2, nl, nr = pl.pallas_call(
        _tab_body,
        grid=(_B,),
        in_specs=[
            pl.BlockSpec((1, _MO, _A), lambda i: (i, 0, 0)),
            pl.BlockSpec((2 * _A, _A), lambda i: (0, 0)),
            pl.BlockSpec((1, _NC * _NID, _NID), lambda i: (i, 0, 0)),
        ],
        out_specs=[
            pl.BlockSpec((1, _MO, _A), lambda i: (i, 0, 0)),
            pl.BlockSpec((1, _MO, _A), lambda i: (i, 0, 0)),
            pl.BlockSpec((1, _NC * _NID, _A), lambda i: (i, 0, 0)),
            pl.BlockSpec((1, _NC * _NID, _A), lambda i: (i, 0, 0)),
        ],
        out_shape=[
            jax.ShapeDtypeStruct((_B, _MO, _A), f32),
            jax.ShapeDtypeStruct((_B, _MO, _A), f32),
            jax.ShapeDtypeStruct((_B, _NC * _NID, _A), f32),
            jax.ShapeDtypeStruct((_B, _NC * _NID, _A), f32),
        ],
    )(attr_feats_lookup, W_attr, nid3)

    i1h = token_pair_idx[..., 0].reshape(_NROW, _CH)
    i2h = token_pair_idx[..., 1].reshape(_NROW, _CH)
    j1h = idx_in_lookup[..., 0].reshape(_NROW, _CH)
    j2h = idx_in_lookup[..., 1].reshape(_NROW, _CH)

    mesh = plsc.VectorSubcoreMesh(core_axis_name="c", subcore_axis_name="s")
    sc = functools.partial(
        pl.kernel,
        out_type=[
            jax.ShapeDtypeStruct((_TOK, _A), f32),
            jax.ShapeDtypeStruct((_TOK, _A), f32),
        ],
        mesh=mesh,
        compiler_params=pltpu.CompilerParams(use_tc_tiling_on_sc=True),
        scratch_types=[
            pltpu.VMEM((_RPW, _CH), jnp.int32),
            pltpu.VMEM((_RPW, _CH), jnp.int32),
            pltpu.VMEM((_RPW, _CH), jnp.int32),
            pltpu.VMEM((_RPW, _CH), jnp.int32),
            pltpu.VMEM((_CH, _A), f32),
            pltpu.VMEM((_CH, _A), f32),
            pltpu.VMEM((_CH, _A), f32),
            pltpu.VMEM((_CH, _A), f32),
            pltpu.SemaphoreType.DMA,
            pltpu.SemaphoreType.DMA,
            pltpu.SemaphoreType.DMA,
            pltpu.SemaphoreType.DMA,
        ],
    )(_sc_body)

    outa, outn = sc(tw1.reshape(_B * _MO, _A),
                    tw2.reshape(_B * _MO, _A),
                    nl.reshape(_NCELL * _NID, _A),
                    nr.reshape(_NCELL * _NID, _A),
                    i1h, i2h, j1h, j2h)

    rmat, fp, smat = pl.pallas_call(
        _prep_body,
        in_specs=[
            pl.BlockSpec((8, 32), lambda: (0, 0)),
            pl.BlockSpec((_A,), lambda: (0,)),
            pl.BlockSpec((32,), lambda: (0,)),
            pl.BlockSpec((32,), lambda: (0,)),
            pl.BlockSpec((32,), lambda: (0,)),
            pl.BlockSpec((3, _OUT), lambda: (0, 0)),
        ],
        out_specs=[
            pl.BlockSpec((12, _OUT), lambda: (0, 0)),
            pl.BlockSpec((2, 64), lambda: (0, 0)),
            pl.BlockSpec((2, 64), lambda: (0, 0)),
        ],
        out_shape=[
            jax.ShapeDtypeStruct((12, _OUT), f32),
            jax.ShapeDtypeStruct((2, 64), f32),
            jax.ShapeDtypeStruct((2, 64), f32),
        ],
    )(W_coor, b_attr, b_coor, basis_freq, phase, type_emb)

    tpt = token_pair_time.reshape(_NCELL, _L, 2)
    tpf = token_pair_time.reshape(_NCELL, 1, _L * 2)
    tty = token_types.reshape(_NCELL, _L, 1)
    cf = coord_feats.reshape(_NCELL, _L, 8)

    ht = pl.pallas_call(
        _time_body,
        grid=(_NCELL,),
        in_specs=[
            pl.BlockSpec((1, _L, 2), lambda i: (i, 0, 0)),
            pl.BlockSpec((1, 1, _L * 2), lambda i: (i, 0, 0)),
            pl.BlockSpec((2, 64), lambda i: (0, 0)),
            pl.BlockSpec((2, 64), lambda i: (0, 0)),
        ],
        out_specs=pl.BlockSpec((1, _L, 64), lambda i: (i, 0, 0)),
        out_shape=jax.ShapeDtypeStruct((_NCELL, _L, 64), f32),
    )(tpt, tpf, fp, smat)

    out = pl.pallas_call(
        _fin_body,
        grid=(_NCELL,),
        in_specs=[
            pl.BlockSpec((1, _L, _A), lambda i: (i, 0, 0)),
            pl.BlockSpec((1, _L, _A), lambda i: (i, 0, 0)),
            pl.BlockSpec((1, _L, 64), lambda i: (i, 0, 0)),
            pl.BlockSpec((1, _L, 1), lambda i: (i, 0, 0)),
            pl.BlockSpec((1, _L, 8), lambda i: (i, 0, 0)),
            pl.BlockSpec((12, _OUT), lambda i: (0, 0)),
        ],
        out_specs=pl.BlockSpec((1, _L, _OUT), lambda i: (i, 0, 0)),
        out_shape=jax.ShapeDtypeStruct((_NCELL, _L, _OUT), f32),
    )(outa.reshape(_NCELL, _L, _A), outn.reshape(_NCELL, _L, _A),
      ht, tty, cf, rmat)

    return out.reshape(_B, _NC, _L, _OUT)


# speedup vs baseline: 1.8972x; 1.8972x over previous
import functools

import jax
import jax.numpy as jnp
from jax import lax
from jax.experimental import pallas as pl
from jax.experimental.pallas import tpu as pltpu
from jax.experimental.pallas import tpu_sc as plsc

_B, _NC, _L = 16, 16, 512
_MO, _A, _NID = 1024, 128, 64
_OUT = 352
_TOK = _B * _NC * _L
_NCELL = _B * _NC
_NCORES, _NSUB = 2, 16
_NW = _NCORES * _NSUB
_CH = 128
_NROW = _TOK // _CH
_RPW = _NROW // _NW


def _tab_body(a_ref, w_ref, nid_ref, tw1_ref, tw2_ref, nl_ref, nr_ref):
    a = a_ref[0]
    w = w_ref[...]
    tw1_ref[0] = jnp.dot(a, w[:_A, :], preferred_element_type=jnp.float32,
                         precision=lax.Precision.HIGHEST)
    tw2_ref[0] = jnp.dot(a, w[_A:, :], preferred_element_type=jnp.float32,
                         precision=lax.Precision.HIGHEST)
    nid = nid_ref[0]
    z = jnp.zeros((_NC * _NID, _NID), jnp.float32)
    nl_ref[0] = jnp.concatenate([nid, z], axis=1)
    nr_ref[0] = jnp.concatenate([z, nid], axis=1)


def _sc_body(tw1, tw2, nl, nr, i1h, i2h, j1h, j2h, outa, outn,
             i1v, i2v, j1v, j2v, g1, g2, h1, h2, s1, s2, s3, s4):
    wid = lax.axis_index("s") * _NCORES + lax.axis_index("c")
    row0 = wid * _RPW
    b = wid // 2

    pltpu.sync_copy(i1h.at[pl.ds(row0, _RPW)], i1v)
    pltpu.sync_copy(i2h.at[pl.ds(row0, _RPW)], i2v)
    pltpu.sync_copy(j1h.at[pl.ds(row0, _RPW)], j1v)
    pltpu.sync_copy(j2h.at[pl.ds(row0, _RPW)], j2v)

    def bias_row(r, carry):
        cell = (row0 + r) // 4
        aoff = b * _MO
        noff = cell * _NID
        for j in range(8):
            sl = pl.ds(j * 16, 16)
            i1v[r, sl] = i1v[r, sl] + aoff
            i2v[r, sl] = i2v[r, sl] + aoff
            j1v[r, sl] = j1v[r, sl] + noff
            j2v[r, sl] = j2v[r, sl] + noff
        return carry

    lax.fori_loop(0, _RPW, bias_row, 0)

    def chunk(c, carry):
        base = (row0 + c) * _CH
        cp1 = pltpu.async_copy(tw1.at[i1v.at[c]], g1, s1)
        cp2 = pltpu.async_copy(tw2.at[i2v.at[c]], g2, s2)
        cp3 = pltpu.async_copy(nl.at[j1v.at[c]], h1, s3)
        cp4 = pltpu.async_copy(nr.at[j2v.at[c]], h2, s4)
        cp1.wait()
        cp2.wait()

        def add_a(r, cc):
            for j in range(8):
                sl = pl.ds(j * 16, 16)
                g1[r, sl] = g1[r, sl] + g2[r, sl]
            return cc

        lax.fori_loop(0, _CH, add_a, 0)
        pltpu.sync_copy(g1, outa.at[pl.ds(base, _CH)])
        cp3.wait()
        cp4.wait()

        def add_n(r, cc):
            for j in range(8):
                sl = pl.ds(j * 16, 16)
                h1[r, sl] = h1[r, sl] + h2[r, sl]
            return cc

        lax.fori_loop(0, _CH, add_n, 0)
        pltpu.sync_copy(h1, outn.at[pl.ds(base, _CH)])
        return carry

    lax.fori_loop(0, _RPW, chunk, 0)


def _prep_body(wc_ref, ba_ref, bc_ref, bf_ref, ph_ref, te_ref,
               rt_ref, ft_ref, pt_ref, st_ref):
    f32 = jnp.float32
    tet = jnp.swapaxes(te_ref[...], 0, 1)
    wct = jnp.swapaxes(wc_ref[...], 0, 1)
    wcol = jnp.concatenate([
        jnp.zeros((128, 8), f32), wct, jnp.zeros((192, 8), f32)], axis=0)
    bcol = jnp.concatenate(
        [ba_ref[...], bc_ref[...], jnp.zeros((192,), f32)]).reshape(_OUT, 1)
    rt_ref[...] = jnp.concatenate([tet, wcol, bcol], axis=1)
    f2 = jnp.concatenate([bf_ref[...], bf_ref[...]]).reshape(64, 1)
    p2 = jnp.concatenate([ph_ref[...], ph_ref[...]]).reshape(64, 1)
    ft_ref[...] = jnp.broadcast_to(f2, (64, _L))
    pt_ref[...] = jnp.broadcast_to(p2, (64, _L))
    lane = lax.broadcasted_iota(jnp.int32, (64, 2), 1)
    row = lax.broadcasted_iota(jnp.int32, (64, 2), 0)
    st_ref[...] = jnp.where((row < 32) == (lane == 0), 1.0, 0.0)


def _time_body(tptt_ref, ft_ref, pt_ref, st_ref, htt_ref):
    tptt = tptt_ref[0]
    maxt = jnp.max(tptt)
    tpts = jnp.dot(st_ref[...], tptt, preferred_element_type=jnp.float32,
                   precision=lax.Precision.HIGHEST)
    htt_ref[...] = jnp.cos((maxt - tpts) * ft_ref[...] + pt_ref[...])


def _fin_body(oa_ref, on_ref, htt_ref, tty_ref, cft_ref, rt_ref, out_ref):
    ttyt = tty_ref[0]
    rows3 = lax.broadcasted_iota(jnp.int32, (3, 1), 0)
    oht = (ttyt == rows3).astype(jnp.float32)
    ones = jnp.ones((1, _L), jnp.float32)
    zt = jnp.concatenate([oht, cft_ref[0], ones], axis=0)
    yt = jnp.dot(rt_ref[...], zt,
                 preferred_element_type=jnp.float32)
    oat = jnp.swapaxes(oa_ref[0], 0, 1)
    ont = jnp.swapaxes(on_ref[0], 0, 1)
    out_ref[0, 0:128, :] = yt[0:128, :] + oat
    out_ref[0, 128:160, :] = yt[128:160, :]
    out_ref[0, 160:224, :] = yt[160:224, :] + htt_ref[...]
    out_ref[0, 224:352, :] = yt[224:352, :] + ont


def kernel(token_pair_idx, token_pair_time, token_types, attr_feats_lookup,
           coord_feats, idx_in_lookup, n_id_lookup,
           W_attr, b_attr, W_coor, b_coor, basis_freq, phase, type_emb):
    f32 = jnp.float32
    nid3 = n_id_lookup.reshape(_B, _NC * _NID, _NID)
    tw1, tw2, nl, nr = pl.pallas_call(
        _tab_body,
        grid=(_B,),
        in_specs=[
            pl.BlockSpec((1, _MO, _A), lambda i: (i, 0, 0)),
            pl.BlockSpec((2 * _A, _A), lambda i: (0, 0)),
            pl.BlockSpec((1, _NC * _NID, _NID), lambda i: (i, 0, 0)),
        ],
        out_specs=[
            pl.BlockSpec((1, _MO, _A), lambda i: (i, 0, 0)),
            pl.BlockSpec((1, _MO, _A), lambda i: (i, 0, 0)),
            pl.BlockSpec((1, _NC * _NID, _A), lambda i: (i, 0, 0)),
            pl.BlockSpec((1, _NC * _NID, _A), lambda i: (i, 0, 0)),
        ],
        out_shape=[
            jax.ShapeDtypeStruct((_B, _MO, _A), f32),
            jax.ShapeDtypeStruct((_B, _MO, _A), f32),
            jax.ShapeDtypeStruct((_B, _NC * _NID, _A), f32),
            jax.ShapeDtypeStruct((_B, _NC * _NID, _A), f32),
        ],
    )(attr_feats_lookup, W_attr, nid3)

    i1h = token_pair_idx[..., 0].reshape(_NROW, _CH)
    i2h = token_pair_idx[..., 1].reshape(_NROW, _CH)
    j1h = idx_in_lookup[..., 0].reshape(_NROW, _CH)
    j2h = idx_in_lookup[..., 1].reshape(_NROW, _CH)

    mesh = plsc.VectorSubcoreMesh(core_axis_name="c", subcore_axis_name="s")
    sc = functools.partial(
        pl.kernel,
        out_type=[
            jax.ShapeDtypeStruct((_TOK, _A), f32),
            jax.ShapeDtypeStruct((_TOK, _A), f32),
        ],
        mesh=mesh,
        scratch_types=[
            pltpu.VMEM((_RPW, _CH), jnp.int32),
            pltpu.VMEM((_RPW, _CH), jnp.int32),
            pltpu.VMEM((_RPW, _CH), jnp.int32),
            pltpu.VMEM((_RPW, _CH), jnp.int32),
            pltpu.VMEM((_CH, _A), f32),
            pltpu.VMEM((_CH, _A), f32),
            pltpu.VMEM((_CH, _A), f32),
            pltpu.VMEM((_CH, _A), f32),
            pltpu.SemaphoreType.DMA,
            pltpu.SemaphoreType.DMA,
            pltpu.SemaphoreType.DMA,
            pltpu.SemaphoreType.DMA,
        ],
    )(_sc_body)

    outa, outn = sc(tw1.reshape(_B * _MO, _A),
                    tw2.reshape(_B * _MO, _A),
                    nl.reshape(_NCELL * _NID, _A),
                    nr.reshape(_NCELL * _NID, _A),
                    i1h, i2h, j1h, j2h)

    rt, ft, pt, st = pl.pallas_call(
        _prep_body,
        in_specs=[
            pl.BlockSpec((8, 32), lambda: (0, 0)),
            pl.BlockSpec((_A,), lambda: (0,)),
            pl.BlockSpec((32,), lambda: (0,)),
            pl.BlockSpec((32,), lambda: (0,)),
            pl.BlockSpec((32,), lambda: (0,)),
            pl.BlockSpec((3, _OUT), lambda: (0, 0)),
        ],
        out_specs=[
            pl.BlockSpec((_OUT, 12), lambda: (0, 0)),
            pl.BlockSpec((64, _L), lambda: (0, 0)),
            pl.BlockSpec((64, _L), lambda: (0, 0)),
            pl.BlockSpec((64, 2), lambda: (0, 0)),
        ],
        out_shape=[
            jax.ShapeDtypeStruct((_OUT, 12), f32),
            jax.ShapeDtypeStruct((64, _L), f32),
            jax.ShapeDtypeStruct((64, _L), f32),
            jax.ShapeDtypeStruct((64, 2), f32),
        ],
    )(W_coor, b_attr, b_coor, basis_freq, phase, type_emb)

    tptt = jnp.swapaxes(token_pair_time.reshape(_NCELL, _L, 2), 1, 2)
    ttyt = token_types.reshape(_NCELL, 1, _L)
    cft = jnp.swapaxes(coord_feats.reshape(_NCELL, _L, 8), 1, 2)

    htt = pl.pallas_call(
        _time_body,
        grid=(_NCELL,),
        in_specs=[
            pl.BlockSpec((1, 2, _L), lambda i: (i, 0, 0)),
            pl.BlockSpec((64, _L), lambda i: (0, 0)),
            pl.BlockSpec((64, _L), lambda i: (0, 0)),
            pl.BlockSpec((64, 2), lambda i: (0, 0)),
        ],
        out_specs=pl.BlockSpec((64, _L), lambda i: (0, i)),
        out_shape=jax.ShapeDtypeStruct((64, _TOK), f32),
    )(tptt, ft, pt, st)

    out = pl.pallas_call(
        _fin_body,
        grid=(_NCELL,),
        in_specs=[
            pl.BlockSpec((1, _L, _A), lambda i: (i, 0, 0)),
            pl.BlockSpec((1, _L, _A), lambda i: (i, 0, 0)),
            pl.BlockSpec((64, _L), lambda i: (0, i)),
            pl.BlockSpec((1, 1, _L), lambda i: (i, 0, 0)),
            pl.BlockSpec((1, 8, _L), lambda i: (i, 0, 0)),
            pl.BlockSpec((_OUT, 12), lambda i: (0, 0)),
        ],
        out_specs=pl.BlockSpec((1, _OUT, _L), lambda i: (i, 0, 0)),
        out_shape=jax.ShapeDtypeStruct((_NCELL, _OUT, _L), f32),
    )(outa.reshape(_NCELL, _L, _A), outn.reshape(_NCELL, _L, _A), htt, ttyt, cft, rt)

    return jnp.swapaxes(out, 1, 2).reshape(_B, _NC, _L, _OUT)


# speedup vs baseline: 2.1865x; 1.1525x over previous
import functools

import jax
import jax.numpy as jnp
from jax import lax
from jax.experimental import pallas as pl
from jax.experimental.pallas import tpu as pltpu
from jax.experimental.pallas import tpu_sc as plsc

_B, _NC, _L = 16, 16, 512
_MO, _A, _NID = 1024, 128, 64
_OUT = 352
_TOK = _B * _NC * _L
_NCELL = _B * _NC
_NCORES, _NSUB = 2, 16
_NW = _NCORES * _NSUB
_CH = 128
_NROW = _TOK // _CH
_NS = 4
_RPS = _NROW // _NS
_CPS = _NCELL // _NS
_RPW = _RPS // _NW


def _tab_body(a_ref, w_ref, nid_ref, tw1_ref, tw2_ref, nl_ref, nr_ref):
    a = a_ref[0]
    w = w_ref[...]
    tw1_ref[0] = jnp.dot(a, w[:_A, :], preferred_element_type=jnp.float32,
                         precision=lax.Precision.HIGHEST)
    tw2_ref[0] = jnp.dot(a, w[_A:, :], preferred_element_type=jnp.float32,
                         precision=lax.Precision.HIGHEST)
    nid = nid_ref[0]
    z = jnp.zeros((_NC * _NID, _NID), jnp.float32)
    nl_ref[0] = jnp.concatenate([nid, z], axis=1)
    nr_ref[0] = jnp.concatenate([z, nid], axis=1)


def _make_sc_body(s):
    def _sc_body(tw1, tw2, nl, nr, i1h, i2h, j1h, j2h, outa, outn,
                 i1v, i2v, j1v, j2v, g1, g2, h1, h2, s1, s2, s3, s4):
        wid = lax.axis_index("s") * _NCORES + lax.axis_index("c")
        lrow0 = wid * _RPW
        grow0 = s * _RPS + lrow0
        b = grow0 // 64

        pltpu.sync_copy(i1h.at[pl.ds(grow0, _RPW)], i1v)
        pltpu.sync_copy(i2h.at[pl.ds(grow0, _RPW)], i2v)
        pltpu.sync_copy(j1h.at[pl.ds(grow0, _RPW)], j1v)
        pltpu.sync_copy(j2h.at[pl.ds(grow0, _RPW)], j2v)

        def bias_row(r, carry):
            cell = (grow0 + r) // 4
            aoff = b * _MO
            noff = cell * _NID
            for j in range(8):
                sl = pl.ds(j * 16, 16)
                i1v[r, sl] = i1v[r, sl] + aoff
                i2v[r, sl] = i2v[r, sl] + aoff
                j1v[r, sl] = j1v[r, sl] + noff
                j2v[r, sl] = j2v[r, sl] + noff
            return carry

        lax.fori_loop(0, _RPW, bias_row, 0)

        def chunk(c, carry):
            base = (lrow0 + c) * _CH
            cp1 = pltpu.async_copy(tw1.at[i1v.at[c]], g1, s1)
            cp2 = pltpu.async_copy(tw2.at[i2v.at[c]], g2, s2)
            cp3 = pltpu.async_copy(nl.at[j1v.at[c]], h1, s3)
            cp4 = pltpu.async_copy(nr.at[j2v.at[c]], h2, s4)
            cp1.wait()
            cp2.wait()

            def add_a(r, cc):
                for j in range(8):
                    sl = pl.ds(j * 16, 16)
                    g1[r, sl] = g1[r, sl] + g2[r, sl]
                return cc

            lax.fori_loop(0, _CH, add_a, 0)
            pltpu.sync_copy(g1, outa.at[pl.ds(base, _CH)])
            cp3.wait()
            cp4.wait()

            def add_n(r, cc):
                for j in range(8):
                    sl = pl.ds(j * 16, 16)
                    h1[r, sl] = h1[r, sl] + h2[r, sl]
                return cc

            lax.fori_loop(0, _CH, add_n, 0)
            pltpu.sync_copy(h1, outn.at[pl.ds(base, _CH)])
            return carry

        lax.fori_loop(0, _RPW, chunk, 0)

    return _sc_body


def _prep_body(wc_ref, ba_ref, bc_ref, bf_ref, ph_ref, te_ref,
               rt_ref, ft_ref, pt_ref, st_ref):
    f32 = jnp.float32
    tet = jnp.swapaxes(te_ref[...], 0, 1)
    wct = jnp.swapaxes(wc_ref[...], 0, 1)
    wcol = jnp.concatenate([
        jnp.zeros((128, 8), f32), wct, jnp.zeros((192, 8), f32)], axis=0)
    bcol = jnp.concatenate(
        [ba_ref[...], bc_ref[...], jnp.zeros((192,), f32)]).reshape(_OUT, 1)
    rt_ref[...] = jnp.concatenate([tet, wcol, bcol], axis=1)
    f2 = jnp.concatenate([bf_ref[...], bf_ref[...]]).reshape(64, 1)
    p2 = jnp.concatenate([ph_ref[...], ph_ref[...]]).reshape(64, 1)
    ft_ref[...] = jnp.broadcast_to(f2, (64, _L))
    pt_ref[...] = jnp.broadcast_to(p2, (64, _L))
    lane = lax.broadcasted_iota(jnp.int32, (64, 2), 1)
    row = lax.broadcasted_iota(jnp.int32, (64, 2), 0)
    st_ref[...] = jnp.where((row < 32) == (lane == 0), 1.0, 0.0)


def _fin_common(oa_ref, on_ref, tptt_ref, tty_ref, cft_ref,
                rt_ref, ft_ref, pt_ref, st_ref, out_ref):
    ttyt = tty_ref[0]
    rows3 = lax.broadcasted_iota(jnp.int32, (3, 1), 0)
    oht = (ttyt == rows3).astype(jnp.float32)
    ones = jnp.ones((1, _L), jnp.float32)
    zt = jnp.concatenate([oht, cft_ref[0], ones], axis=0)
    yt = jnp.dot(rt_ref[...], zt,
                 preferred_element_type=jnp.float32)
    tptt = tptt_ref[0]
    maxt = jnp.max(tptt)
    tpts = jnp.dot(st_ref[...], tptt, preferred_element_type=jnp.float32,
                   precision=lax.Precision.HIGHEST)
    h = jnp.cos((maxt - tpts) * ft_ref[...] + pt_ref[...])
    oat = jnp.swapaxes(oa_ref[0], 0, 1)
    ont = jnp.swapaxes(on_ref[0], 0, 1)
    out_ref[0, 0:128, :] = yt[0:128, :] + oat
    out_ref[0, 128:160, :] = yt[128:160, :]
    out_ref[0, 160:224, :] = yt[160:224, :] + h
    out_ref[0, 224:352, :] = yt[224:352, :] + ont


def _fin_body0(oa_ref, on_ref, tptt_ref, tty_ref, cft_ref,
               rt_ref, ft_ref, pt_ref, st_ref, out_ref):
    _fin_common(oa_ref, on_ref, tptt_ref, tty_ref, cft_ref,
                rt_ref, ft_ref, pt_ref, st_ref, out_ref)


def _fin_body1(oa_ref, on_ref, tptt_ref, tty_ref, cft_ref,
               rt_ref, ft_ref, pt_ref, st_ref, prev_ref, out_ref):
    del prev_ref
    _fin_common(oa_ref, on_ref, tptt_ref, tty_ref, cft_ref,
                rt_ref, ft_ref, pt_ref, st_ref, out_ref)


def kernel(token_pair_idx, token_pair_time, token_types, attr_feats_lookup,
           coord_feats, idx_in_lookup, n_id_lookup,
           W_attr, b_attr, W_coor, b_coor, basis_freq, phase, type_emb):
    f32 = jnp.float32
    nid3 = n_id_lookup.reshape(_B, _NC * _NID, _NID)
    tw1, tw2, nl, nr = pl.pallas_call(
        _tab_body,
        grid=(_B,),
        in_specs=[
            pl.BlockSpec((1, _MO, _A), lambda i: (i, 0, 0)),
            pl.BlockSpec((2 * _A, _A), lambda i: (0, 0)),
            pl.BlockSpec((1, _NC * _NID, _NID), lambda i: (i, 0, 0)),
        ],
        out_specs=[
            pl.BlockSpec((1, _MO, _A), lambda i: (i, 0, 0)),
            pl.BlockSpec((1, _MO, _A), lambda i: (i, 0, 0)),
            pl.BlockSpec((1, _NC * _NID, _A), lambda i: (i, 0, 0)),
            pl.BlockSpec((1, _NC * _NID, _A), lambda i: (i, 0, 0)),
        ],
        out_shape=[
            jax.ShapeDtypeStruct((_B, _MO, _A), f32),
            jax.ShapeDtypeStruct((_B, _MO, _A), f32),
            jax.ShapeDtypeStruct((_B, _NC * _NID, _A), f32),
            jax.ShapeDtypeStruct((_B, _NC * _NID, _A), f32),
        ],
    )(attr_feats_lookup, W_attr, nid3)

    i1h = token_pair_idx[..., 0].reshape(_NROW, _CH)
    i2h = token_pair_idx[..., 1].reshape(_NROW, _CH)
    j1h = idx_in_lookup[..., 0].reshape(_NROW, _CH)
    j2h = idx_in_lookup[..., 1].reshape(_NROW, _CH)
    tw1f = tw1.reshape(_B * _MO, _A)
    tw2f = tw2.reshape(_B * _MO, _A)
    nlf = nl.reshape(_NCELL * _NID, _A)
    nrf = nr.reshape(_NCELL * _NID, _A)

    mesh = plsc.VectorSubcoreMesh(core_axis_name="c", subcore_axis_name="s")
    strips = []
    for s in range(_NS):
        sc = functools.partial(
            pl.kernel,
            out_type=[
                jax.ShapeDtypeStruct((_TOK // _NS, _A), f32),
                jax.ShapeDtypeStruct((_TOK // _NS, _A), f32),
            ],
            mesh=mesh,
            compiler_params=pltpu.CompilerParams(use_tc_tiling_on_sc=True),
            scratch_types=[
                pltpu.VMEM((_RPW, _CH), jnp.int32),
                pltpu.VMEM((_RPW, _CH), jnp.int32),
                pltpu.VMEM((_RPW, _CH), jnp.int32),
                pltpu.VMEM((_RPW, _CH), jnp.int32),
                pltpu.VMEM((_CH, _A), f32),
                pltpu.VMEM((_CH, _A), f32),
                pltpu.VMEM((_CH, _A), f32),
                pltpu.VMEM((_CH, _A), f32),
                pltpu.SemaphoreType.DMA,
                pltpu.SemaphoreType.DMA,
                pltpu.SemaphoreType.DMA,
                pltpu.SemaphoreType.DMA,
            ],
        )(_make_sc_body(s))
        strips.append(sc(tw1f, tw2f, nlf, nrf, i1h, i2h, j1h, j2h))

    rt, ft, pt, st = pl.pallas_call(
        _prep_body,
        in_specs=[
            pl.BlockSpec((8, 32), lambda: (0, 0)),
            pl.BlockSpec((_A,), lambda: (0,)),
            pl.BlockSpec((32,), lambda: (0,)),
            pl.BlockSpec((32,), lambda: (0,)),
            pl.BlockSpec((32,), lambda: (0,)),
            pl.BlockSpec((3, _OUT), lambda: (0, 0)),
        ],
        out_specs=[
            pl.BlockSpec((_OUT, 12), lambda: (0, 0)),
            pl.BlockSpec((64, _L), lambda: (0, 0)),
            pl.BlockSpec((64, _L), lambda: (0, 0)),
            pl.BlockSpec((64, 2), lambda: (0, 0)),
        ],
        out_shape=[
            jax.ShapeDtypeStruct((_OUT, 12), f32),
            jax.ShapeDtypeStruct((64, _L), f32),
            jax.ShapeDtypeStruct((64, _L), f32),
            jax.ShapeDtypeStruct((64, 2), f32),
        ],
    )(W_coor, b_attr, b_coor, basis_freq, phase, type_emb)

    tptt = jnp.swapaxes(token_pair_time.reshape(_NCELL, _L, 2), 1, 2)
    ttyt = token_types.reshape(_NCELL, 1, _L)
    cft = jnp.swapaxes(coord_feats.reshape(_NCELL, _L, 8), 1, 2)

    out = None
    for s in range(_NS):
        outa, outn = strips[s]
        common_in = [
            pl.BlockSpec((1, _L, _A), lambda i: (i, 0, 0)),
            pl.BlockSpec((1, _L, _A), lambda i: (i, 0, 0)),
            pl.BlockSpec((1, 2, _L), lambda i, s=s: (i + _CPS * s, 0, 0)),
            pl.BlockSpec((1, 1, _L), lambda i, s=s: (i + _CPS * s, 0, 0)),
            pl.BlockSpec((1, 8, _L), lambda i, s=s: (i + _CPS * s, 0, 0)),
            pl.BlockSpec((_OUT, 12), lambda i: (0, 0)),
            pl.BlockSpec((64, _L), lambda i: (0, 0)),
            pl.BlockSpec((64, _L), lambda i: (0, 0)),
            pl.BlockSpec((64, 2), lambda i: (0, 0)),
        ]
        args = [outa.reshape(_CPS, _L, _A), outn.reshape(_CPS, _L, _A),
                tptt, ttyt, cft, rt, ft, pt, st]
        if s == 0:
            out = pl.pallas_call(
                _fin_body0,
                grid=(_CPS,),
                in_specs=common_in,
                out_specs=pl.BlockSpec((1, _OUT, _L),
                                       lambda i, s=s: (i + _CPS * s, 0, 0)),
                out_shape=jax.ShapeDtypeStruct((_NCELL, _OUT, _L), f32),
            )(*args)
        else:
            out = pl.pallas_call(
                _fin_body1,
                grid=(_CPS,),
                in_specs=common_in + [pl.BlockSpec(memory_space=pl.ANY)],
                out_specs=pl.BlockSpec((1, _OUT, _L),
                                       lambda i, s=s: (i + _CPS * s, 0, 0)),
                out_shape=jax.ShapeDtypeStruct((_NCELL, _OUT, _L), f32),
                input_output_aliases={9: 0},
            )(*args, out)

    return jnp.swapaxes(out, 1, 2).reshape(_B, _NC, _L, _OUT)


# speedup vs baseline: 2.3632x; 1.0808x over previous
import functools

import jax
import jax.numpy as jnp
from jax import lax
from jax.experimental import pallas as pl
from jax.experimental.pallas import tpu as pltpu
from jax.experimental.pallas import tpu_sc as plsc

_B, _NC, _L = 16, 16, 512
_MO, _A, _NID = 1024, 128, 64
_OUT = 352
_TOK = _B * _NC * _L
_NCELL = _B * _NC
_NCORES, _NSUB = 2, 16
_NW = _NCORES * _NSUB
_CH = 128
_NROW = _TOK // _CH
_NS = 4
_RPS = _NROW // _NS
_CPS = _NCELL // _NS
_RPW = _RPS // _NW


def _tab_body(a_ref, w_ref, nid_ref, tw1_ref, tw2_ref, nl_ref, nr_ref):
    a = a_ref[0]
    w = w_ref[...]
    tw1_ref[0] = jnp.dot(a, w[:_A, :], preferred_element_type=jnp.float32,
                         precision=lax.Precision.HIGHEST)
    tw2_ref[0] = jnp.dot(a, w[_A:, :], preferred_element_type=jnp.float32,
                         precision=lax.Precision.HIGHEST)
    nid = nid_ref[0]
    z = jnp.zeros((_NC * _NID, _NID), jnp.float32)
    nl_ref[0] = jnp.concatenate([nid, z], axis=1)
    nr_ref[0] = jnp.concatenate([z, nid], axis=1)


def _make_sc_body(s):
    def _sc_body(tw1, tw2, nl, nr, i1h, i2h, j1h, j2h, outa, outn,
                 i1v, i2v, j1v, j2v, g1, g2, h1, h2, s1, s2, s3, s4):
        wid = lax.axis_index("s") * _NCORES + lax.axis_index("c")
        lrow0 = wid * _RPW
        grow0 = s * _RPS + lrow0
        b = grow0 // 64

        pltpu.sync_copy(i1h.at[pl.ds(grow0, _RPW)], i1v)
        pltpu.sync_copy(i2h.at[pl.ds(grow0, _RPW)], i2v)
        pltpu.sync_copy(j1h.at[pl.ds(grow0, _RPW)], j1v)
        pltpu.sync_copy(j2h.at[pl.ds(grow0, _RPW)], j2v)

        def bias_row(r, carry):
            cell = (grow0 + r) // 4
            aoff = b * _MO
            noff = cell * _NID
            for j in range(8):
                sl = pl.ds(j * 16, 16)
                i1v[r, sl] = i1v[r, sl] + aoff
                i2v[r, sl] = i2v[r, sl] + aoff
                j1v[r, sl] = j1v[r, sl] + noff
                j2v[r, sl] = j2v[r, sl] + noff
            return carry

        lax.fori_loop(0, _RPW, bias_row, 0)

        def chunk(c, carry):
            base = (lrow0 + c) * _CH
            cp1 = pltpu.async_copy(tw1.at[i1v.at[c]], g1, s1)
            cp2 = pltpu.async_copy(tw2.at[i2v.at[c]], g2, s2)
            cp3 = pltpu.async_copy(nl.at[j1v.at[c]], h1, s3)
            cp4 = pltpu.async_copy(nr.at[j2v.at[c]], h2, s4)
            cp1.wait()
            cp2.wait()

            def add_a(r, cc):
                for j in range(8):
                    sl = pl.ds(j * 16, 16)
                    g1[r, sl] = g1[r, sl] + g2[r, sl]
                return cc

            lax.fori_loop(0, _CH, add_a, 0)
            pltpu.sync_copy(g1, outa.at[pl.ds(base, _CH)])
            cp3.wait()
            cp4.wait()

            def add_n(r, cc):
                for j in range(8):
                    sl = pl.ds(j * 16, 16)
                    h1[r, sl] = h1[r, sl] + h2[r, sl]
                return cc

            lax.fori_loop(0, _CH, add_n, 0)
            pltpu.sync_copy(h1, outn.at[pl.ds(base, _CH)])
            return carry

        lax.fori_loop(0, _RPW, chunk, 0)

    return _sc_body


def _prep_body(wc_ref, ba_ref, bc_ref, bf_ref, ph_ref, te_ref,
               rt_ref, ft_ref, pt_ref, st_ref):
    f32 = jnp.float32
    tet = jnp.swapaxes(te_ref[...], 0, 1)
    wct = jnp.swapaxes(wc_ref[...], 0, 1)
    wcol = jnp.concatenate([
        jnp.zeros((128, 8), f32), wct, jnp.zeros((192, 8), f32)], axis=0)
    bcol = jnp.concatenate(
        [ba_ref[...], bc_ref[...], jnp.zeros((192,), f32)]).reshape(_OUT, 1)
    rt_ref[...] = jnp.concatenate([tet, wcol, bcol], axis=1)
    inv2pi = 1.0 / (2.0 * jnp.pi)
    f2 = jnp.concatenate([bf_ref[...], bf_ref[...]]).reshape(64, 1) * inv2pi
    p2 = jnp.concatenate([ph_ref[...], ph_ref[...]]).reshape(64, 1) * inv2pi
    ft_ref[...] = jnp.broadcast_to(f2, (64, _L))
    pt_ref[...] = jnp.broadcast_to(p2, (64, _L))
    lane = lax.broadcasted_iota(jnp.int32, (64, 2), 1)
    row = lax.broadcasted_iota(jnp.int32, (64, 2), 0)
    st_ref[...] = jnp.where((row < 32) == (lane == 0), 1.0, 0.0)


def _fin_common(oa_ref, on_ref, tptt_ref, tty_ref, cft_ref,
                rt_ref, ft_ref, pt_ref, st_ref, out_ref):
    ttyt = tty_ref[0]
    rows3 = lax.broadcasted_iota(jnp.int32, (3, 1), 0)
    oht = (ttyt == rows3).astype(jnp.float32)
    ones = jnp.ones((1, _L), jnp.float32)
    zt = jnp.concatenate([oht, cft_ref[0], ones], axis=0)
    yt = jnp.dot(rt_ref[...], zt,
                 preferred_element_type=jnp.float32)
    tptt = tptt_ref[0]
    maxt = jnp.max(tptt)
    tpts = jnp.dot(st_ref[...], tptt, preferred_element_type=jnp.float32,
                   precision=lax.Precision.HIGHEST)
    r = (maxt - tpts) * ft_ref[...] + pt_ref[...]
    u = r - jnp.round(r)
    z = u * u
    h = -1.7143907110886711
    for c in (7.9035363713184648, -26.426256783374388, 60.244641371876639,
              -85.456817206693714, 64.939394022668282, -19.739208802178716,
              1.0):
        h = h * z + c
    oat = jnp.swapaxes(oa_ref[0], 0, 1)
    ont = jnp.swapaxes(on_ref[0], 0, 1)
    out_ref[0, 0:128, :] = yt[0:128, :] + oat
    out_ref[0, 128:160, :] = yt[128:160, :]
    out_ref[0, 160:224, :] = yt[160:224, :] + h
    out_ref[0, 224:352, :] = yt[224:352, :] + ont


def _fin_body0(oa_ref, on_ref, tptt_ref, tty_ref, cft_ref,
               rt_ref, ft_ref, pt_ref, st_ref, out_ref):
    _fin_common(oa_ref, on_ref, tptt_ref, tty_ref, cft_ref,
                rt_ref, ft_ref, pt_ref, st_ref, out_ref)


def _fin_body1(oa_ref, on_ref, tptt_ref, tty_ref, cft_ref,
               rt_ref, ft_ref, pt_ref, st_ref, prev_ref, out_ref):
    del prev_ref
    _fin_common(oa_ref, on_ref, tptt_ref, tty_ref, cft_ref,
                rt_ref, ft_ref, pt_ref, st_ref, out_ref)


def kernel(token_pair_idx, token_pair_time, token_types, attr_feats_lookup,
           coord_feats, idx_in_lookup, n_id_lookup,
           W_attr, b_attr, W_coor, b_coor, basis_freq, phase, type_emb):
    f32 = jnp.float32
    nid3 = n_id_lookup.reshape(_B, _NC * _NID, _NID)
    tw1, tw2, nl, nr = pl.pallas_call(
        _tab_body,
        grid=(_B,),
        in_specs=[
            pl.BlockSpec((1, _MO, _A), lambda i: (i, 0, 0)),
            pl.BlockSpec((2 * _A, _A), lambda i: (0, 0)),
            pl.BlockSpec((1, _NC * _NID, _NID), lambda i: (i, 0, 0)),
        ],
        out_specs=[
            pl.BlockSpec((1, _MO, _A), lambda i: (i, 0, 0)),
            pl.BlockSpec((1, _MO, _A), lambda i: (i, 0, 0)),
            pl.BlockSpec((1, _NC * _NID, _A), lambda i: (i, 0, 0)),
            pl.BlockSpec((1, _NC * _NID, _A), lambda i: (i, 0, 0)),
        ],
        out_shape=[
            jax.ShapeDtypeStruct((_B, _MO, _A), f32),
            jax.ShapeDtypeStruct((_B, _MO, _A), f32),
            jax.ShapeDtypeStruct((_B, _NC * _NID, _A), f32),
            jax.ShapeDtypeStruct((_B, _NC * _NID, _A), f32),
        ],
    )(attr_feats_lookup, W_attr, nid3)

    i1h = token_pair_idx[..., 0].reshape(_NROW, _CH)
    i2h = token_pair_idx[..., 1].reshape(_NROW, _CH)
    j1h = idx_in_lookup[..., 0].reshape(_NROW, _CH)
    j2h = idx_in_lookup[..., 1].reshape(_NROW, _CH)
    tw1f = tw1.reshape(_B * _MO, _A)
    tw2f = tw2.reshape(_B * _MO, _A)
    nlf = nl.reshape(_NCELL * _NID, _A)
    nrf = nr.reshape(_NCELL * _NID, _A)

    mesh = plsc.VectorSubcoreMesh(core_axis_name="c", subcore_axis_name="s")
    strips = []
    for s in range(_NS):
        sc = functools.partial(
            pl.kernel,
            out_type=[
                jax.ShapeDtypeStruct((_TOK // _NS, _A), f32),
                jax.ShapeDtypeStruct((_TOK // _NS, _A), f32),
            ],
            mesh=mesh,
            compiler_params=pltpu.CompilerParams(use_tc_tiling_on_sc=True),
            scratch_types=[
                pltpu.VMEM((_RPW, _CH), jnp.int32),
                pltpu.VMEM((_RPW, _CH), jnp.int32),
                pltpu.VMEM((_RPW, _CH), jnp.int32),
                pltpu.VMEM((_RPW, _CH), jnp.int32),
                pltpu.VMEM((_CH, _A), f32),
                pltpu.VMEM((_CH, _A), f32),
                pltpu.VMEM((_CH, _A), f32),
                pltpu.VMEM((_CH, _A), f32),
                pltpu.SemaphoreType.DMA,
                pltpu.SemaphoreType.DMA,
                pltpu.SemaphoreType.DMA,
                pltpu.SemaphoreType.DMA,
            ],
        )(_make_sc_body(s))
        strips.append(sc(tw1f, tw2f, nlf, nrf, i1h, i2h, j1h, j2h))

    rt, ft, pt, st = pl.pallas_call(
        _prep_body,
        in_specs=[
            pl.BlockSpec((8, 32), lambda: (0, 0)),
            pl.BlockSpec((_A,), lambda: (0,)),
            pl.BlockSpec((32,), lambda: (0,)),
            pl.BlockSpec((32,), lambda: (0,)),
            pl.BlockSpec((32,), lambda: (0,)),
            pl.BlockSpec((3, _OUT), lambda: (0, 0)),
        ],
        out_specs=[
            pl.BlockSpec((_OUT, 12), lambda: (0, 0)),
            pl.BlockSpec((64, _L), lambda: (0, 0)),
            pl.BlockSpec((64, _L), lambda: (0, 0)),
            pl.BlockSpec((64, 2), lambda: (0, 0)),
        ],
        out_shape=[
            jax.ShapeDtypeStruct((_OUT, 12), f32),
            jax.ShapeDtypeStruct((64, _L), f32),
            jax.ShapeDtypeStruct((64, _L), f32),
            jax.ShapeDtypeStruct((64, 2), f32),
        ],
    )(W_coor, b_attr, b_coor, basis_freq, phase, type_emb)

    tptt = jnp.swapaxes(token_pair_time.reshape(_NCELL, _L, 2), 1, 2)
    ttyt = token_types.reshape(_NCELL, 1, _L)
    cft = jnp.swapaxes(coord_feats.reshape(_NCELL, _L, 8), 1, 2)

    out = None
    for s in range(_NS):
        outa, outn = strips[s]
        common_in = [
            pl.BlockSpec((1, _L, _A), lambda i: (i, 0, 0)),
            pl.BlockSpec((1, _L, _A), lambda i: (i, 0, 0)),
            pl.BlockSpec((1, 2, _L), lambda i, s=s: (i + _CPS * s, 0, 0)),
            pl.BlockSpec((1, 1, _L), lambda i, s=s: (i + _CPS * s, 0, 0)),
            pl.BlockSpec((1, 8, _L), lambda i, s=s: (i + _CPS * s, 0, 0)),
            pl.BlockSpec((_OUT, 12), lambda i: (0, 0)),
            pl.BlockSpec((64, _L), lambda i: (0, 0)),
            pl.BlockSpec((64, _L), lambda i: (0, 0)),
            pl.BlockSpec((64, 2), lambda i: (0, 0)),
        ]
        args = [outa.reshape(_CPS, _L, _A), outn.reshape(_CPS, _L, _A),
                tptt, ttyt, cft, rt, ft, pt, st]
        if s == 0:
            out = pl.pallas_call(
                _fin_body0,
                grid=(_CPS,),
                in_specs=common_in,
                out_specs=pl.BlockSpec((1, _OUT, _L),
                                       lambda i, s=s: (i + _CPS * s, 0, 0)),
                out_shape=jax.ShapeDtypeStruct((_NCELL, _OUT, _L), f32),
            )(*args)
        else:
            out = pl.pallas_call(
                _fin_body1,
                grid=(_CPS,),
                in_specs=common_in + [pl.BlockSpec(memory_space=pl.ANY)],
                out_specs=pl.BlockSpec((1, _OUT, _L),
                                       lambda i, s=s: (i + _CPS * s, 0, 0)),
                out_shape=jax.ShapeDtypeStruct((_NCELL, _OUT, _L), f32),
                input_output_aliases={9: 0},
            )(*args, out)

    return jnp.swapaxes(out, 1, 2).reshape(_B, _NC, _L, _OUT)
